# baseline (speedup 1.0000x reference)
import functools

import jax
import jax.numpy as jnp
from jax import lax
from jax.experimental import pallas as pl
from jax.experimental.pallas import tpu as pltpu
from jax.experimental.pallas import tpu_sc as plsc

N = 320000
D = 128
H = 32
S = 512
ROWW = 144

BA = 4000


def _head_body(x_ref, w1_ref, b1_ref, w2_ref, b2_ref, e_ref):
    h = jnp.tanh(
        jnp.dot(x_ref[...], w1_ref[...], preferred_element_type=jnp.float32)
        + b1_ref[...]
    )
    u = jnp.sum(jnp.abs(w2_ref[...])) + jnp.abs(b2_ref[0, 0])
    lt = lax.dot_general(w2_ref[...], h, (((1,), (1,)), ((), ())),
                         preferred_element_type=jnp.float32)
    e_ref[...] = jnp.exp(lt + (b2_ref[0, 0] - u)).reshape(1, 1, BA)


def _head(x, W1, b1r, w2r, b2r):
    return pl.pallas_call(
        _head_body,
        grid=(N // BA,),
        in_specs=[
            pl.BlockSpec((BA, D), lambda i: (i, 0)),
            pl.BlockSpec((D, H), lambda i: (0, 0)),
            pl.BlockSpec((1, H), lambda i: (0, 0)),
            pl.BlockSpec((1, H), lambda i: (0, 0)),
            pl.BlockSpec((1, 1), lambda i: (0, 0)),
        ],
        out_specs=pl.BlockSpec((1, 1, BA), lambda i: (i, 0, 0)),
        out_shape=jax.ShapeDtypeStruct((N // BA, 1, BA), jnp.float32),
    )(x, W1, b1r, w2r, b2r)


NC, NS = 2, 16
NW = NC * NS
R = N // NW
C = 80
NCHUNK = R // C
SEG_PER_TILE = S // NS


def _sc_body(x_hbm, ew_hbm, batch_hbm, part_hbm, xv0, xv1, xe_v, ewb, bb,
             z_v, fl_v, fl1, idx1, acc_sh, sx0, sx1):
    cid = lax.axis_index("c")
    sid = lax.axis_index("s")
    wid = sid * NC + cid
    base = wid * R
    mbase = wid * NCHUNK

    def zrow(i, carry):
        for j in range(ROWW // 16):
            z_v[i, pl.ds(j * 16, 16)] = jnp.zeros((16,), jnp.float32)
        return carry
    lax.fori_loop(0, SEG_PER_TILE, zrow, 0)

    def zfl(i, carry):
        for j in range(ROWW // 16):
            fl_v[i, pl.ds(j * 16, 16)] = jnp.zeros((16,), jnp.float32)
        return carry
    lax.fori_loop(0, 16, zfl, 0)
    pltpu.sync_copy(z_v, acc_sh.at[pl.ds(sid * SEG_PER_TILE, SEG_PER_TILE)])
    pltpu.sync_copy(ew_hbm.at[pl.ds(mbase, NCHUNK)], ewb)
    pltpu.sync_copy(batch_hbm.at[pl.ds(mbase, NCHUNK)], bb)
    plsc.subcore_barrier()

    def zv16():
        return jnp.zeros((16,), jnp.float32)

    def issue(k, xv, sem):
        pltpu.async_copy(x_hbm.at[pl.ds(base + k * C, C), :], xv, sem)

    def stage_flush_row(accs, seg):
        for j in range(ROWW // 16):
            fl1[pl.ds(j * 16, 16)] = accs[j]
        for j in range(ROWW // 16):
            fl_v[0, pl.ds(j * 16, 16)] = fl1[pl.ds(j * 16, 16)]
        idx1[...] = jnp.full((16,), seg, jnp.int32)

    def do_chunk(k, xv, sem, accs):
        pltpu.make_async_copy(
            x_hbm.at[pl.ds(base + k * C, C), :], xv, sem).wait()
        first = bb[k, pl.ds(0, 16)][0]
        last = bb[k, pl.ds(C - 16, 16)][15]
        km1 = jnp.maximum(k - 1, 0)
        z01 = jnp.minimum(k, 1)
        prev = (bb[km1, pl.ds(C - 16, 16)][15] * z01 + first * (1 - z01))
        uniform = first == last
        do_flush = jnp.logical_or(first != last, first != prev)

        stage_flush_row(accs, prev)

        @pl.when(do_flush)
        def _():
            pltpu.sync_copy(fl_v, acc_sh.at[idx1], add=True)

        keep = jnp.where(do_flush, 0.0, 1.0).astype(jnp.float32)
        u = jnp.where(uniform, 1.0, 0.0).astype(jnp.float32)

        def acc_body(g, a):
            ev16 = ewb[k, pl.ds(g * 16, 16)]
            i0 = g * 16
            a = list(a)
            for r in range(16):
                for j in range(D // 16):
                    xe = xv[i0 + r, pl.ds(j * 16, 16)] * ev16[r]
                    xe_v[i0 + r, pl.ds(j * 16, 16)] = xe
                    a[j] = a[j] + xe
                m0 = lax.iota(jnp.int32, 16) == 0
                xe_v[i0 + r, pl.ds(D, 16)] = jnp.where(
                    m0, zv16() + ev16[r], zv16())
            a[D // 16] = a[D // 16] + ev16
            return tuple(a)
        accs_new = plsc.parallel_loop(
            0, C // 16, unroll=C // 16,
            carry=tuple(a * keep for a in accs))(acc_body)

        @pl.when(jnp.logical_not(uniform))
        def _():
            pltpu.sync_copy(xe_v, acc_sh.at[bb.at[k]], add=True)

        return tuple(a * u for a in accs_new)

    issue(0, xv0, sx0)
    issue(1, xv1, sx1)
    carry0 = tuple(zv16() for _ in range(9))

    def pair(kk, carry):
        k0 = 2 * kk
        carry = do_chunk(k0, xv0, sx0, carry)
        issue(k0 + 2, xv0, sx0)
        k1 = 2 * kk + 1
        carry = do_chunk(k1, xv1, sx1, carry)

        @pl.when(kk < (NCHUNK - 3) // 2)
        def _():
            issue(k1 + 2, xv1, sx1)
        return carry
    carryf = lax.fori_loop(0, (NCHUNK - 1) // 2, pair, carry0)
    carryf = do_chunk(NCHUNK - 1, xv0, sx0, carryf)
    stage_flush_row(carryf, bb[NCHUNK - 1, pl.ds(C - 16, 16)][15])
    pltpu.sync_copy(fl_v, acc_sh.at[idx1], add=True)
    plsc.subcore_barrier()

    pltpu.sync_copy(acc_sh.at[pl.ds(sid * SEG_PER_TILE, SEG_PER_TILE)],
                    part_hbm.at[cid, pl.ds(sid * SEG_PER_TILE, SEG_PER_TILE)])


def _sc_reduce(x, ew2d, batch2d):
    call = pl.kernel(
        _sc_body,
        out_type=jax.ShapeDtypeStruct((NC, S, ROWW), jnp.float32),
        mesh=plsc.VectorSubcoreMesh(core_axis_name="c", subcore_axis_name="s"),
        scratch_types=[
            pltpu.VMEM((C, D), jnp.float32),
            pltpu.VMEM((C, D), jnp.float32),
            pltpu.VMEM((C, ROWW), jnp.float32),
            pltpu.VMEM((NCHUNK, C), jnp.float32),
            pltpu.VMEM((NCHUNK, C), jnp.int32),
            pltpu.VMEM((SEG_PER_TILE, ROWW), jnp.float32),
            pltpu.VMEM((16, ROWW), jnp.float32),
            pltpu.VMEM((ROWW,), jnp.float32),
            pltpu.VMEM((16,), jnp.int32),
            pltpu.VMEM_SHARED((S, ROWW), jnp.float32),
            pltpu.SemaphoreType.DMA,
            pltpu.SemaphoreType.DMA,
        ],
        compiler_params=pltpu.CompilerParams(use_tc_tiling_on_sc=False),
    )
    return call(x, ew2d, batch2d)


def _combine_body(p_ref, o_ref):
    p = p_ref[0] + p_ref[1]
    num = p[:, :D]
    den = jnp.sum(p[:, D:], axis=1, keepdims=True)
    o_ref[...] = num / jnp.maximum(den, 1e-12)


def _combine(part):
    return pl.pallas_call(
        _combine_body,
        out_shape=jax.ShapeDtypeStruct((S, D), jnp.float32),
    )(part)


def kernel(x, batch, W1, b1, W2, b2):
    b1r = b1.reshape(1, H)
    w2r = W2.reshape(1, H)
    b2r = b2.reshape(1, 1)
    e3 = _head(x, W1, b1r, w2r, b2r)
    ew2d = e3.reshape(N // C, C)
    batch2d = batch.reshape(N // C, C)
    part = _sc_reduce(x, ew2d, batch2d)
    return _combine(part)


# speedup vs baseline: 2.2011x; 2.2011x over previous
import functools

import jax
import jax.numpy as jnp
from jax import lax
from jax.experimental import pallas as pl
from jax.experimental.pallas import tpu as pltpu
from jax.experimental.pallas import tpu_sc as plsc

N = 320000
D = 128
H = 32
S = 512
ROWW = 144

BA = 4000


def _head_body(x_ref, w1_ref, b1_ref, w2_ref, b2_ref, e_ref):
    h = jnp.tanh(
        jnp.dot(x_ref[...], w1_ref[...], preferred_element_type=jnp.float32)
        + b1_ref[...]
    )
    u = jnp.sum(jnp.abs(w2_ref[...])) + jnp.abs(b2_ref[0, 0])
    lt = lax.dot_general(w2_ref[...], h, (((1,), (1,)), ((), ())),
                         preferred_element_type=jnp.float32)
    e_ref[...] = jnp.exp(lt + (b2_ref[0, 0] - u)).reshape(1, 1, BA)


def _head(x, W1, b1r, w2r, b2r):
    return pl.pallas_call(
        _head_body,
        grid=(N // BA,),
        in_specs=[
            pl.BlockSpec((BA, D), lambda i: (i, 0)),
            pl.BlockSpec((D, H), lambda i: (0, 0)),
            pl.BlockSpec((1, H), lambda i: (0, 0)),
            pl.BlockSpec((1, H), lambda i: (0, 0)),
            pl.BlockSpec((1, 1), lambda i: (0, 0)),
        ],
        out_specs=pl.BlockSpec((1, 1, BA), lambda i: (i, 0, 0)),
        out_shape=jax.ShapeDtypeStruct((N // BA, 1, BA), jnp.float32),
    )(x, W1, b1r, w2r, b2r)


NC, NS = 2, 16
NW = NC * NS
R = N // NW
C = 80
NCHUNK = R // C
SEG_PER_TILE = S // NS


def _sc_body(x_hbm, ew_hbm, batch_hbm, part_hbm, xv0, xv1, xe_v, ewb, bb,
             z_v, fl_v, fl1, idx1, acc_sh, sx0, sx1):
    cid = lax.axis_index("c")
    sid = lax.axis_index("s")
    wid = sid * NC + cid
    base = wid * R
    mbase = wid * NCHUNK

    def zrow(i, carry):
        for j in range(ROWW // 16):
            z_v[i, pl.ds(j * 16, 16)] = jnp.zeros((16,), jnp.float32)
        return carry
    lax.fori_loop(0, SEG_PER_TILE, zrow, 0)

    def zfl(i, carry):
        for j in range(ROWW // 16):
            fl_v[i, pl.ds(j * 16, 16)] = jnp.zeros((16,), jnp.float32)
        return carry
    lax.fori_loop(0, 16, zfl, 0)
    pltpu.sync_copy(z_v, acc_sh.at[pl.ds(sid * SEG_PER_TILE, SEG_PER_TILE)])
    pltpu.sync_copy(ew_hbm.at[pl.ds(mbase, NCHUNK)], ewb)
    pltpu.sync_copy(batch_hbm.at[pl.ds(mbase, NCHUNK)], bb)
    plsc.subcore_barrier()

    def zv16():
        return jnp.zeros((16,), jnp.float32)

    def issue(k, xv, sem):
        pltpu.async_copy(x_hbm.at[pl.ds(base + k * C, C), :], xv, sem)

    def stage_flush_row(accs, seg):
        for j in range(ROWW // 16):
            fl1[pl.ds(j * 16, 16)] = accs[j]
        for j in range(ROWW // 16):
            fl_v[0, pl.ds(j * 16, 16)] = fl1[pl.ds(j * 16, 16)]
        idx1[...] = jnp.full((16,), seg, jnp.int32)

    def do_chunk(k, xv, sem, accs):
        pltpu.make_async_copy(
            x_hbm.at[pl.ds(base + k * C, C), :], xv, sem).wait()
        first = bb[k, pl.ds(0, 16)][0]
        last = bb[k, pl.ds(C - 16, 16)][15]
        km1 = jnp.maximum(k - 1, 0)
        z01 = jnp.minimum(k, 1)
        prev = (bb[km1, pl.ds(C - 16, 16)][15] * z01 + first * (1 - z01))
        uniform = first == last
        do_flush = jnp.logical_or(first != last, first != prev)

        stage_flush_row(accs, prev)

        @pl.when(do_flush)
        def _():
            pltpu.sync_copy(fl_v, acc_sh.at[idx1], add=True)

        keep = jnp.where(do_flush, 0.0, 1.0).astype(jnp.float32)
        u = jnp.where(uniform, 1.0, 0.0).astype(jnp.float32)

        def acc_body(g, a):
            ev16 = ewb[k, pl.ds(g * 16, 16)]
            i0 = g * 16
            a = list(a)
            for r in range(16):
                for j in range(D // 16):
                    a[j] = a[j] + xv[i0 + r, pl.ds(j * 16, 16)] * ev16[r]
            a[D // 16] = a[D // 16] + ev16
            return tuple(a)
        accs_new = plsc.parallel_loop(
            0, C // 16, unroll=C // 16,
            carry=tuple(a * keep for a in accs))(acc_body)

        @pl.when(jnp.logical_not(uniform))
        def _():
            @plsc.parallel_loop(0, C // 16, unroll=C // 16)
            def grp(g):
                ev16 = ewb[k, pl.ds(g * 16, 16)]
                i0 = g * 16
                for r in range(16):
                    for j in range(D // 16):
                        xe_v[i0 + r, pl.ds(j * 16, 16)] = (
                            xv[i0 + r, pl.ds(j * 16, 16)] * ev16[r])
                    m0 = lax.iota(jnp.int32, 16) == 0
                    xe_v[i0 + r, pl.ds(D, 16)] = jnp.where(
                        m0, zv16() + ev16[r], zv16())
            pltpu.sync_copy(xe_v, acc_sh.at[bb.at[k]], add=True)

        return tuple(a * u for a in accs_new)

    issue(0, xv0, sx0)
    issue(1, xv1, sx1)
    carry0 = tuple(zv16() for _ in range(9))

    def pair(kk, carry):
        k0 = 2 * kk
        carry = do_chunk(k0, xv0, sx0, carry)
        issue(k0 + 2, xv0, sx0)
        k1 = 2 * kk + 1
        carry = do_chunk(k1, xv1, sx1, carry)

        @pl.when(kk < (NCHUNK - 3) // 2)
        def _():
            issue(k1 + 2, xv1, sx1)
        return carry
    carryf = lax.fori_loop(0, (NCHUNK - 1) // 2, pair, carry0)
    carryf = do_chunk(NCHUNK - 1, xv0, sx0, carryf)
    stage_flush_row(carryf, bb[NCHUNK - 1, pl.ds(C - 16, 16)][15])
    pltpu.sync_copy(fl_v, acc_sh.at[idx1], add=True)
    plsc.subcore_barrier()

    pltpu.sync_copy(acc_sh.at[pl.ds(sid * SEG_PER_TILE, SEG_PER_TILE)],
                    part_hbm.at[cid, pl.ds(sid * SEG_PER_TILE, SEG_PER_TILE)])


def _sc_reduce(x, ew2d, batch2d):
    call = pl.kernel(
        _sc_body,
        out_type=jax.ShapeDtypeStruct((NC, S, ROWW), jnp.float32),
        mesh=plsc.VectorSubcoreMesh(core_axis_name="c", subcore_axis_name="s"),
        scratch_types=[
            pltpu.VMEM((C, D), jnp.float32),
            pltpu.VMEM((C, D), jnp.float32),
            pltpu.VMEM((C, ROWW), jnp.float32),
            pltpu.VMEM((NCHUNK, C), jnp.float32),
            pltpu.VMEM((NCHUNK, C), jnp.int32),
            pltpu.VMEM((SEG_PER_TILE, ROWW), jnp.float32),
            pltpu.VMEM((16, ROWW), jnp.float32),
            pltpu.VMEM((ROWW,), jnp.float32),
            pltpu.VMEM((16,), jnp.int32),
            pltpu.VMEM_SHARED((S, ROWW), jnp.float32),
            pltpu.SemaphoreType.DMA,
            pltpu.SemaphoreType.DMA,
        ],
        compiler_params=pltpu.CompilerParams(use_tc_tiling_on_sc=False),
    )
    return call(x, ew2d, batch2d)


def _combine_body(p_ref, o_ref):
    p = p_ref[0] + p_ref[1]
    num = p[:, :D]
    den = jnp.sum(p[:, D:], axis=1, keepdims=True)
    o_ref[...] = num / jnp.maximum(den, 1e-12)


def _combine(part):
    return pl.pallas_call(
        _combine_body,
        out_shape=jax.ShapeDtypeStruct((S, D), jnp.float32),
    )(part)


def kernel(x, batch, W1, b1, W2, b2):
    b1r = b1.reshape(1, H)
    w2r = W2.reshape(1, H)
    b2r = b2.reshape(1, 1)
    e3 = _head(x, W1, b1r, w2r, b2r)
    ew2d = e3.reshape(N // C, C)
    batch2d = batch.reshape(N // C, C)
    part = _sc_reduce(x, ew2d, batch2d)
    return _combine(part)
